# baseline (speedup 1.0000x reference)
import jax
import jax.numpy as jnp
from jax import lax
from jax.experimental import pallas as pl
from jax.experimental.pallas import tpu as pltpu
from jax.experimental.pallas import tpu_sc as plsc

N_NODES = 50000
IN_FEATS = 512
NUM_GRAPHS = 256
NW = 32
PER_W = 1568
N_PAD = NW * PER_W
CHUNK = 96
L = 16
NJ = IN_FEATS // L
BLOCK = 1000
NUM_BLOCKS = N_NODES // BLOCK
ZROWS = 16


def _matvec_body(f_ref, w_ref, b_ref, aw_ref, sig_ref):
    aw = jax.lax.dot_general(
        f_ref[...], w_ref[...], (((1,), (0,)), ((), ())),
        preferred_element_type=jnp.float32,
    ) + b_ref[0, 0]
    aw_ref[...] = aw
    sig_ref[...] = jax.nn.sigmoid(aw)


def _tc_matvec(feats, W, b2):
    return pl.pallas_call(
        _matvec_body,
        grid=(NUM_BLOCKS,),
        in_specs=[
            pl.BlockSpec((BLOCK, IN_FEATS), lambda i: (i, 0)),
            pl.BlockSpec((IN_FEATS, 1), lambda i: (0, 0)),
            pl.BlockSpec((1, 1), lambda i: (0, 0)),
        ],
        out_specs=[
            pl.BlockSpec((BLOCK, 1), lambda i: (i, 0)),
            pl.BlockSpec((BLOCK, 1), lambda i: (i, 0)),
        ],
        out_shape=[
            jax.ShapeDtypeStruct((N_NODES, 1), jnp.float32),
            jax.ShapeDtypeStruct((N_NODES, 1), jnp.float32),
        ],
    )(feats, W, b2)


def _flush(accs, acc_buf, parts, region, cur):
    for j in range(NJ):
        acc_buf[pl.ds(j * L, L)] = accs[j]
    pltpu.sync_copy(acc_buf, parts.at[pl.ds(region + cur * IN_FEATS,
                                            IN_FEATS)])


def _process_rows(n_groups, loc, carry, row_buf, ids_buf, w_buf,
                  acc_buf, parts, region):
    nrows = n_groups * L

    def rbody(r, carry):
        cur = carry[0]
        accs = carry[1:]
        s_r = ids_buf[pl.ds(loc + r, L)][0]
        w_r = w_buf[pl.ds(loc + r, L)][0]
        changed = s_r != cur

        @pl.when(changed)
        def _():
            _flush(accs, acc_buf, parts, region, cur)

        new_accs = tuple(
            jnp.where(changed,
                      w_r * row_buf[r, pl.ds(j * L, L)],
                      accs[j] + w_r * row_buf[r, pl.ds(j * L, L)])
            for j in range(NJ))
        return (s_r,) + new_accs

    return lax.fori_loop(0, nrows, rbody, carry, unroll=4)


def _sc_body(feats2d, ids, wvals, parts,
             row_buf0, row_buf1, ids_buf, w_buf, acc_buf, zero_buf,
             sem0, sem1):
    cid = lax.axis_index("c")
    sid = lax.axis_index("s")
    wid = cid * 16 + sid
    region = wid * NUM_GRAPHS * IN_FEATS

    z = jnp.zeros((L,), jnp.float32)
    for j in range(ZROWS * NJ):
        zero_buf[pl.ds(j * L, L)] = z
    for k in range(NUM_GRAPHS // ZROWS):
        pltpu.sync_copy(
            zero_buf,
            parts.at[pl.ds(region + k * ZROWS * IN_FEATS, ZROWS * IN_FEATS)])

    start = wid * PER_W
    count = jnp.minimum(PER_W, N_NODES - start)
    nfull = count // CHUNK
    ntail = (count - nfull * CHUNK) // L

    pltpu.sync_copy(ids.at[pl.ds(start, PER_W)], ids_buf.at[pl.ds(0, PER_W)])
    pltpu.sync_copy(wvals.at[pl.ds(start, PER_W)], w_buf.at[pl.ds(0, PER_W)])

    zero = jnp.zeros((L,), jnp.float32)
    carry0 = (jnp.int32(0),) + (zero,) * NJ

    def dma_start(buf, sem, k):
        pltpu.async_copy(feats2d.at[pl.ds(start + k * CHUNK, CHUNK)], buf,
                         sem)

    def dma_wait(buf, sem):
        pltpu.make_async_copy(feats2d.at[pl.ds(0, CHUNK)], buf, sem).wait()

    def proc(buf, k, n_groups, carry):
        return _process_rows(n_groups, k * CHUNK, carry, buf, ids_buf,
                             w_buf, acc_buf, parts, region)

    dma_start(row_buf0, sem0, 0)

    def pair_body(k2, carry):
        k = 2 * k2
        dma_wait(row_buf0, sem0)
        dma_start(row_buf1, sem1, k + 1)
        carry = proc(row_buf0, k, CHUNK // L, carry)
        dma_wait(row_buf1, sem1)

        @pl.when(k + 2 < nfull)
        def _():
            dma_start(row_buf0, sem0, k + 2)

        return proc(row_buf1, k + 1, CHUNK // L, carry)

    carry = lax.cond(nfull > 0,
                     lambda c: lax.fori_loop(0, nfull // 2, pair_body, c),
                     lambda c: c, carry0)

    def tail_body(k, carry):
        pos = start + nfull * CHUNK + k * L
        pltpu.sync_copy(feats2d.at[pl.ds(pos, L)], row_buf0.at[pl.ds(0, L)])
        return _process_rows(1, pos - start, carry,
                             row_buf0, ids_buf, w_buf, acc_buf, parts,
                             region)

    carry = lax.fori_loop(0, ntail, tail_body, carry)

    _flush(carry[1:], acc_buf, parts, region, carry[0])


@jax.jit
def _sc_call(feats2d, ids_pad, w_pad):
    mesh = plsc.VectorSubcoreMesh(core_axis_name="c", subcore_axis_name="s",
                                  num_cores=2, num_subcores=16)
    return pl.kernel(
        _sc_body,
        out_type=jax.ShapeDtypeStruct((NW * NUM_GRAPHS * IN_FEATS,),
                                      jnp.float32),
        mesh=mesh,
        compiler_params=pltpu.CompilerParams(use_tc_tiling_on_sc=True,
                                             needs_layout_passes=False),
        scratch_types=[
            pltpu.VMEM((CHUNK, IN_FEATS), jnp.float32),
            pltpu.VMEM((CHUNK, IN_FEATS), jnp.float32),
            pltpu.VMEM((PER_W + L,), jnp.int32),
            pltpu.VMEM((PER_W + L,), jnp.float32),
            pltpu.VMEM((IN_FEATS,), jnp.float32),
            pltpu.VMEM((ZROWS * IN_FEATS,), jnp.float32),
            pltpu.SemaphoreType.DMA,
            pltpu.SemaphoreType.DMA,
        ],
    )(feats2d, ids_pad, w_pad)


def _merge_body(p_ref, out_ref):
    i = pl.program_id(0)

    @pl.when(i == 0)
    def _():
        out_ref[...] = p_ref[0]

    @pl.when(i > 0)
    def _():
        out_ref[...] += p_ref[0]


def kernel(feats, segment_ids, W, b):
    b2 = b.reshape(1, 1).astype(jnp.float32)
    aw, wv = _tc_matvec(feats, W, b2)
    ids_pad = jnp.pad(segment_ids.astype(jnp.int32), (0, N_PAD - N_NODES),
                      constant_values=NUM_GRAPHS - 1)
    w_pad = jnp.pad(wv.reshape(N_NODES), (0, N_PAD - N_NODES))
    parts = _sc_call(feats, ids_pad, w_pad)
    hg = pl.pallas_call(
        _merge_body,
        grid=(NW,),
        in_specs=[pl.BlockSpec((1, NUM_GRAPHS, IN_FEATS), lambda i: (i, 0, 0))],
        out_specs=pl.BlockSpec((NUM_GRAPHS, IN_FEATS), lambda i: (0, 0)),
        out_shape=jax.ShapeDtypeStruct((NUM_GRAPHS, IN_FEATS), jnp.float32),
    )(parts.reshape(NW, NUM_GRAPHS, IN_FEATS))
    return (hg, aw)


# speedup vs baseline: 1.0944x; 1.0944x over previous
import jax
import jax.numpy as jnp
from jax import lax
from jax.experimental import pallas as pl
from jax.experimental.pallas import tpu as pltpu
from jax.experimental.pallas import tpu_sc as plsc

N_NODES = 50000
IN_FEATS = 512
NUM_GRAPHS = 256
NW = 32
PER_W = 1568
N_PAD = NW * PER_W
CHUNK = 96
L = 16
NJ = IN_FEATS // L
BLOCK = 5000
NUM_BLOCKS = N_NODES // BLOCK
ZROWS = 16


def _matvec_body(f_ref, w_ref, b_ref, aw_ref, sig_ref):
    aw = jax.lax.dot_general(
        f_ref[...], w_ref[...], (((1,), (0,)), ((), ())),
        preferred_element_type=jnp.float32,
    ) + b_ref[0, 0]
    aw_ref[...] = aw
    sig_ref[...] = jax.nn.sigmoid(aw)


def _tc_matvec(feats, W, b2):
    return pl.pallas_call(
        _matvec_body,
        grid=(NUM_BLOCKS,),
        in_specs=[
            pl.BlockSpec((BLOCK, IN_FEATS), lambda i: (i, 0)),
            pl.BlockSpec((IN_FEATS, 1), lambda i: (0, 0)),
            pl.BlockSpec((1, 1), lambda i: (0, 0)),
        ],
        out_specs=[
            pl.BlockSpec((BLOCK, 1), lambda i: (i, 0)),
            pl.BlockSpec((BLOCK, 1), lambda i: (i, 0)),
        ],
        out_shape=[
            jax.ShapeDtypeStruct((N_NODES, 1), jnp.float32),
            jax.ShapeDtypeStruct((N_NODES, 1), jnp.float32),
        ],
    )(feats, W, b2)


def _flush(accs, acc_buf, parts, region, cur):
    for j in range(NJ):
        acc_buf[pl.ds(j * L, L)] = accs[j]
    pltpu.sync_copy(acc_buf, parts.at[pl.ds(region + cur * IN_FEATS,
                                            IN_FEATS)])


def _process_rows(n_groups, loc, carry, row_buf, ids_buf, w_buf,
                  acc_buf, parts, region):
    nrows = n_groups * L

    def rbody(r, carry):
        cur = carry[0]
        accs = carry[1:]
        s_r = ids_buf[pl.ds(loc + r, L)][0]
        w_r = w_buf[pl.ds(loc + r, L)][0]
        changed = s_r != cur

        @pl.when(changed)
        def _():
            _flush(accs, acc_buf, parts, region, cur)

        new_accs = tuple(
            jnp.where(changed,
                      w_r * row_buf[r, pl.ds(j * L, L)],
                      accs[j] + w_r * row_buf[r, pl.ds(j * L, L)])
            for j in range(NJ))
        return (s_r,) + new_accs

    return lax.fori_loop(0, nrows, rbody, carry, unroll=4)


def _sc_body(feats2d, ids, wvals, parts,
             row_buf0, row_buf1, ids_buf, w_buf, acc_buf, zero_buf,
             sem0, sem1):
    cid = lax.axis_index("c")
    sid = lax.axis_index("s")
    wid = cid * 16 + sid
    region = wid * NUM_GRAPHS * IN_FEATS

    z = jnp.zeros((L,), jnp.float32)
    for j in range(ZROWS * NJ):
        zero_buf[pl.ds(j * L, L)] = z
    for k in range(NUM_GRAPHS // ZROWS):
        pltpu.sync_copy(
            zero_buf,
            parts.at[pl.ds(region + k * ZROWS * IN_FEATS, ZROWS * IN_FEATS)])

    start = wid * PER_W
    count = jnp.minimum(PER_W, N_NODES - start)
    nfull = count // CHUNK
    ntail = (count - nfull * CHUNK) // L

    pltpu.sync_copy(ids.at[pl.ds(start, PER_W)], ids_buf.at[pl.ds(0, PER_W)])
    pltpu.sync_copy(wvals.at[pl.ds(start, PER_W)], w_buf.at[pl.ds(0, PER_W)])

    zero = jnp.zeros((L,), jnp.float32)
    carry0 = (jnp.int32(0),) + (zero,) * NJ

    def dma_start(buf, sem, k):
        pltpu.async_copy(feats2d.at[pl.ds(start + k * CHUNK, CHUNK)], buf,
                         sem)

    def dma_wait(buf, sem):
        pltpu.make_async_copy(feats2d.at[pl.ds(0, CHUNK)], buf, sem).wait()

    def proc(buf, k, n_groups, carry):
        return _process_rows(n_groups, k * CHUNK, carry, buf, ids_buf,
                             w_buf, acc_buf, parts, region)

    dma_start(row_buf0, sem0, 0)

    def pair_body(k2, carry):
        k = 2 * k2
        dma_wait(row_buf0, sem0)
        dma_start(row_buf1, sem1, k + 1)
        carry = proc(row_buf0, k, CHUNK // L, carry)
        dma_wait(row_buf1, sem1)

        @pl.when(k + 2 < nfull)
        def _():
            dma_start(row_buf0, sem0, k + 2)

        return proc(row_buf1, k + 1, CHUNK // L, carry)

    carry = lax.cond(nfull > 0,
                     lambda c: lax.fori_loop(0, nfull // 2, pair_body, c),
                     lambda c: c, carry0)

    def tail_body(k, carry):
        pos = start + nfull * CHUNK + k * L
        pltpu.sync_copy(feats2d.at[pl.ds(pos, L)], row_buf0.at[pl.ds(0, L)])
        return _process_rows(1, pos - start, carry,
                             row_buf0, ids_buf, w_buf, acc_buf, parts,
                             region)

    carry = lax.fori_loop(0, ntail, tail_body, carry)

    _flush(carry[1:], acc_buf, parts, region, carry[0])


@jax.jit
def _sc_call(feats2d, ids_pad, w_pad):
    mesh = plsc.VectorSubcoreMesh(core_axis_name="c", subcore_axis_name="s",
                                  num_cores=2, num_subcores=16)
    return pl.kernel(
        _sc_body,
        out_type=jax.ShapeDtypeStruct((NW * NUM_GRAPHS * IN_FEATS,),
                                      jnp.float32),
        mesh=mesh,
        compiler_params=pltpu.CompilerParams(use_tc_tiling_on_sc=True,
                                             needs_layout_passes=False),
        scratch_types=[
            pltpu.VMEM((CHUNK, IN_FEATS), jnp.float32),
            pltpu.VMEM((CHUNK, IN_FEATS), jnp.float32),
            pltpu.VMEM((PER_W + L,), jnp.int32),
            pltpu.VMEM((PER_W + L,), jnp.float32),
            pltpu.VMEM((IN_FEATS,), jnp.float32),
            pltpu.VMEM((ZROWS * IN_FEATS,), jnp.float32),
            pltpu.SemaphoreType.DMA,
            pltpu.SemaphoreType.DMA,
        ],
    )(feats2d, ids_pad, w_pad)


def _merge_body(p_ref, out_ref):
    i = pl.program_id(0)

    @pl.when(i == 0)
    def _():
        out_ref[...] = p_ref[0]

    @pl.when(i > 0)
    def _():
        out_ref[...] += p_ref[0]


def kernel(feats, segment_ids, W, b):
    b2 = b.reshape(1, 1).astype(jnp.float32)
    aw, wv = _tc_matvec(feats, W, b2)
    ids_pad = jnp.pad(segment_ids.astype(jnp.int32), (0, N_PAD - N_NODES),
                      constant_values=NUM_GRAPHS - 1)
    w_pad = jnp.pad(wv.reshape(N_NODES), (0, N_PAD - N_NODES))
    parts = _sc_call(feats, ids_pad, w_pad)
    hg = pl.pallas_call(
        _merge_body,
        grid=(NW,),
        in_specs=[pl.BlockSpec((1, NUM_GRAPHS, IN_FEATS), lambda i: (i, 0, 0))],
        out_specs=pl.BlockSpec((NUM_GRAPHS, IN_FEATS), lambda i: (0, 0)),
        out_shape=jax.ShapeDtypeStruct((NUM_GRAPHS, IN_FEATS), jnp.float32),
    )(parts.reshape(NW, NUM_GRAPHS, IN_FEATS))
    return (hg, aw)


# speedup vs baseline: 1.1051x; 1.0098x over previous
import jax
import jax.numpy as jnp
from jax import lax
from jax.experimental import pallas as pl
from jax.experimental.pallas import tpu as pltpu
from jax.experimental.pallas import tpu_sc as plsc

N_NODES = 50000
IN_FEATS = 512
NUM_GRAPHS = 256
NW = 32
PER_W = 1568
N_PAD = NW * PER_W
CHUNK = 96
L = 16
NJ = IN_FEATS // L
BLOCK = 5000
NUM_BLOCKS = N_NODES // BLOCK
ZROWS = 16


def _matvec_body(f_ref, w_ref, b_ref, aw_ref, sig_ref):
    aw = jax.lax.dot_general(
        f_ref[...], w_ref[...], (((1,), (0,)), ((), ())),
        preferred_element_type=jnp.float32,
    ) + b_ref[0, 0]
    aw_ref[...] = aw
    sig_ref[...] = jax.nn.sigmoid(aw)


def _tc_matvec(feats, W, b2):
    return pl.pallas_call(
        _matvec_body,
        grid=(NUM_BLOCKS,),
        in_specs=[
            pl.BlockSpec((BLOCK, IN_FEATS), lambda i: (i, 0)),
            pl.BlockSpec((IN_FEATS, 1), lambda i: (0, 0)),
            pl.BlockSpec((1, 1), lambda i: (0, 0)),
        ],
        out_specs=[
            pl.BlockSpec((BLOCK, 1), lambda i: (i, 0)),
            pl.BlockSpec((BLOCK, 1), lambda i: (i, 0)),
        ],
        out_shape=[
            jax.ShapeDtypeStruct((N_NODES, 1), jnp.float32),
            jax.ShapeDtypeStruct((N_NODES, 1), jnp.float32),
        ],
    )(feats, W, b2)


def _flush(accs, acc_buf, parts, region, cur):
    for j in range(NJ):
        acc_buf[pl.ds(j * L, L)] = accs[j]
    pltpu.sync_copy(acc_buf, parts.at[pl.ds(region + cur * IN_FEATS,
                                            IN_FEATS)])


def _process_rows(n_groups, loc, carry, row_buf, ids_buf, w_buf,
                  acc_buf, parts, region):
    nrows = n_groups * L

    def rbody(r, carry):
        cur = carry[0]
        accs = carry[1:]
        s_r = ids_buf[pl.ds(loc + r, L)][0]
        w_r = w_buf[pl.ds(loc + r, L)][0]
        changed = s_r != cur

        @pl.when(changed)
        def _():
            _flush(accs, acc_buf, parts, region, cur)

        new_accs = tuple(
            jnp.where(changed,
                      w_r * row_buf[r, pl.ds(j * L, L)],
                      accs[j] + w_r * row_buf[r, pl.ds(j * L, L)])
            for j in range(NJ))
        return (s_r,) + new_accs

    return lax.fori_loop(0, nrows, rbody, carry, unroll=8)


def _sc_body(feats2d, ids, wvals, parts,
             row_buf0, row_buf1, ids_buf, w_buf, acc_buf, zero_buf,
             sem0, sem1):
    cid = lax.axis_index("c")
    sid = lax.axis_index("s")
    wid = cid * 16 + sid
    region = wid * NUM_GRAPHS * IN_FEATS

    z = jnp.zeros((L,), jnp.float32)
    for j in range(ZROWS * NJ):
        zero_buf[pl.ds(j * L, L)] = z
    for k in range(NUM_GRAPHS // ZROWS):
        pltpu.sync_copy(
            zero_buf,
            parts.at[pl.ds(region + k * ZROWS * IN_FEATS, ZROWS * IN_FEATS)])

    start = wid * PER_W
    count = jnp.minimum(PER_W, N_NODES - start)
    nfull = count // CHUNK
    ntail = (count - nfull * CHUNK) // L

    pltpu.sync_copy(ids.at[pl.ds(start, PER_W)], ids_buf.at[pl.ds(0, PER_W)])
    pltpu.sync_copy(wvals.at[pl.ds(start, PER_W)], w_buf.at[pl.ds(0, PER_W)])

    zero = jnp.zeros((L,), jnp.float32)
    carry0 = (jnp.int32(0),) + (zero,) * NJ

    def dma_start(buf, sem, k):
        pltpu.async_copy(feats2d.at[pl.ds(start + k * CHUNK, CHUNK)], buf,
                         sem)

    def dma_wait(buf, sem):
        pltpu.make_async_copy(feats2d.at[pl.ds(0, CHUNK)], buf, sem).wait()

    def proc(buf, k, n_groups, carry):
        return _process_rows(n_groups, k * CHUNK, carry, buf, ids_buf,
                             w_buf, acc_buf, parts, region)

    dma_start(row_buf0, sem0, 0)

    def pair_body(k2, carry):
        k = 2 * k2
        dma_wait(row_buf0, sem0)
        dma_start(row_buf1, sem1, k + 1)
        carry = proc(row_buf0, k, CHUNK // L, carry)
        dma_wait(row_buf1, sem1)

        @pl.when(k + 2 < nfull)
        def _():
            dma_start(row_buf0, sem0, k + 2)

        return proc(row_buf1, k + 1, CHUNK // L, carry)

    carry = lax.cond(nfull > 0,
                     lambda c: lax.fori_loop(0, nfull // 2, pair_body, c),
                     lambda c: c, carry0)

    def tail_body(k, carry):
        pos = start + nfull * CHUNK + k * L
        pltpu.sync_copy(feats2d.at[pl.ds(pos, L)], row_buf0.at[pl.ds(0, L)])
        return _process_rows(1, pos - start, carry,
                             row_buf0, ids_buf, w_buf, acc_buf, parts,
                             region)

    carry = lax.fori_loop(0, ntail, tail_body, carry)

    _flush(carry[1:], acc_buf, parts, region, carry[0])


@jax.jit
def _sc_call(feats2d, ids_pad, w_pad):
    mesh = plsc.VectorSubcoreMesh(core_axis_name="c", subcore_axis_name="s",
                                  num_cores=2, num_subcores=16)
    return pl.kernel(
        _sc_body,
        out_type=jax.ShapeDtypeStruct((NW * NUM_GRAPHS * IN_FEATS,),
                                      jnp.float32),
        mesh=mesh,
        compiler_params=pltpu.CompilerParams(use_tc_tiling_on_sc=True,
                                             needs_layout_passes=False),
        scratch_types=[
            pltpu.VMEM((CHUNK, IN_FEATS), jnp.float32),
            pltpu.VMEM((CHUNK, IN_FEATS), jnp.float32),
            pltpu.VMEM((PER_W + L,), jnp.int32),
            pltpu.VMEM((PER_W + L,), jnp.float32),
            pltpu.VMEM((IN_FEATS,), jnp.float32),
            pltpu.VMEM((ZROWS * IN_FEATS,), jnp.float32),
            pltpu.SemaphoreType.DMA,
            pltpu.SemaphoreType.DMA,
        ],
    )(feats2d, ids_pad, w_pad)


def _merge_body(p_ref, out_ref):
    i = pl.program_id(0)

    @pl.when(i == 0)
    def _():
        out_ref[...] = p_ref[0]

    @pl.when(i > 0)
    def _():
        out_ref[...] += p_ref[0]


def kernel(feats, segment_ids, W, b):
    b2 = b.reshape(1, 1).astype(jnp.float32)
    aw, wv = _tc_matvec(feats, W, b2)
    ids_pad = jnp.pad(segment_ids.astype(jnp.int32), (0, N_PAD - N_NODES),
                      constant_values=NUM_GRAPHS - 1)
    w_pad = jnp.pad(wv.reshape(N_NODES), (0, N_PAD - N_NODES))
    parts = _sc_call(feats, ids_pad, w_pad)
    hg = pl.pallas_call(
        _merge_body,
        grid=(NW,),
        in_specs=[pl.BlockSpec((1, NUM_GRAPHS, IN_FEATS), lambda i: (i, 0, 0))],
        out_specs=pl.BlockSpec((NUM_GRAPHS, IN_FEATS), lambda i: (0, 0)),
        out_shape=jax.ShapeDtypeStruct((NUM_GRAPHS, IN_FEATS), jnp.float32),
    )(parts.reshape(NW, NUM_GRAPHS, IN_FEATS))
    return (hg, aw)
